# baseline (speedup 1.0000x reference)
import functools

import jax
import jax.numpy as jnp
from jax import lax
from jax.experimental import pallas as pl
from jax.experimental.pallas import tpu as pltpu
from jax.experimental.pallas import tpu_sc as plsc

_B = 16384
_D = 64
_NC = 2
_NS = 16
_NW = _NC * _NS
_CHUNK = _B // _NW
_LAG = 3
_SLOTS = 4


def _als_body(uid_hbm, iid_hbm, ut_hbm, it_hbm, ub_hbm, ib_hbm, out_hbm,
              uid_v, iid_v, fidx_v, ubuf, ibuf, ub_v, ib_v, out_v,
              sem_ids, sem_b, sem_u0, sem_u1, sem_u2, sem_u3,
              sem_i0, sem_i1, sem_i2, sem_i3):
    wid = lax.axis_index("s") * _NC + lax.axis_index("c")
    base = wid * _CHUNK

    cp_uid = pltpu.async_copy(uid_hbm.at[pl.ds(base, _CHUNK)], uid_v, sem_ids)
    cp_iid = pltpu.async_copy(iid_hbm.at[pl.ds(base, _CHUNK)], iid_v, sem_ids)
    for c in range(_D // 16):
        fidx_v[pl.ds(c * 16, 16)] = lax.iota(jnp.int32, 16) + c * 16
    cp_uid.wait()
    cp_iid.wait()

    cp_ub = pltpu.async_copy(ub_hbm.at[uid_v], ub_v, sem_b)
    cp_ib = pltpu.async_copy(ib_hbm.at[iid_v], ib_v, sem_b)
    cp_ub.wait()
    cp_ib.wait()

    lanes = lax.iota(jnp.int32, 16)
    lanes16 = [lanes + 16 * c for c in range(_D // 16)]
    sems_u = (sem_u0, sem_u1, sem_u2, sem_u3)
    sems_i = (sem_i0, sem_i1, sem_i2, sem_i3)

    def fire(slot, ru, ri):
        ro = pl.multiple_of((ru >> 7) * 128, 128)
        so = pl.multiple_of((ri >> 7) * 128, 128)
        pltpu.async_copy(ut_hbm.at[fidx_v, pl.ds(ro, 128)],
                         ubuf.at[slot], sems_u[slot])
        pltpu.async_copy(it_hbm.at[fidx_v, pl.ds(so, 128)],
                         ibuf.at[slot], sems_i[slot])

    def drain(slot, ru, ri):
        ro = pl.multiple_of((ru >> 7) * 128, 128)
        so = pl.multiple_of((ri >> 7) * 128, 128)
        pltpu.make_async_copy(ut_hbm.at[fidx_v, pl.ds(ro, 128)],
                              ubuf.at[slot], sems_u[slot]).wait()
        pltpu.make_async_copy(it_hbm.at[fidx_v, pl.ds(so, 128)],
                              ibuf.at[slot], sems_i[slot]).wait()


    def group(g, carry):
        goff = pl.multiple_of(g * 16, 16)
        noff = pl.multiple_of(jnp.minimum(g + 1, _CHUNK // 16 - 1) * 16, 16)
        ucur = uid_v[pl.ds(goff, 16)]
        icur = iid_v[pl.ds(goff, 16)]
        unext = uid_v[pl.ds(noff, 16)]
        inext = iid_v[pl.ds(noff, 16)]
        tot = jnp.zeros((16,), jnp.float32)
        cps = {}

        def fire2(j):
            slot = j & (_SLOTS - 1)
            ro = pl.multiple_of((ucur[j] >> 7) * 128, 128)
            so = pl.multiple_of((icur[j] >> 7) * 128, 128)
            cu = pltpu.async_copy(ut_hbm.at[:, pl.ds(ro, 128)],
                                  ubuf.at[slot], sems_u[slot])
            ci = pltpu.async_copy(it_hbm.at[:, pl.ds(so, 128)],
                                  ibuf.at[slot], sems_i[slot])
            cps[j] = (cu, ci)

        for j in range(_LAG):
            fire2(j)
        for j in range(16):
            b = g * 16 + j
            if j + _LAG < 16:
                fire2(j + _LAG)
            cu, ci = cps.pop(j)
            cu.wait()
            ci.wait()
            slot16 = jnp.full((16,), j & (_SLOTS - 1), jnp.int32)
            cu16 = jnp.full((16,), ucur[j] & 127, jnp.int32)
            ci16 = jnp.full((16,), icur[j] & 127, jnp.int32)
            acc = jnp.zeros((16,), jnp.float32)
            for c in range(_D // 16):
                gu = plsc.load_gather(ubuf, [slot16, lanes16[c], cu16])
                gi = plsc.load_gather(ibuf, [slot16, lanes16[c], ci16])
                acc = acc + gu * gi
            tot = jnp.where(lanes == j, jnp.sum(acc), tot)
        out_v[pl.ds(goff, 16)] = (tot + ub_v[pl.ds(goff, 16)]
                                  + ib_v[pl.ds(goff, 16)])
        return carry

    lax.fori_loop(0, _CHUNK // 16, group, 0)

    pltpu.sync_copy(out_v, out_hbm.at[pl.ds(base, _CHUNK)])


_als = functools.partial(
    pl.kernel,
    out_type=jax.ShapeDtypeStruct((_B,), jnp.float32),
    mesh=plsc.VectorSubcoreMesh(core_axis_name="c", subcore_axis_name="s",
                                num_cores=_NC, num_subcores=_NS),
    compiler_params=pltpu.CompilerParams(needs_layout_passes=False,
                                         use_tc_tiling_on_sc=True),
    scratch_types=[
        pltpu.VMEM((_CHUNK,), jnp.int32),
        pltpu.VMEM((_CHUNK,), jnp.int32),
        pltpu.VMEM((_D,), jnp.int32),
        pltpu.VMEM((_SLOTS, _D, 128), jnp.float32),
        pltpu.VMEM((_SLOTS, _D, 128), jnp.float32),
        pltpu.VMEM((_CHUNK,), jnp.float32),
        pltpu.VMEM((_CHUNK,), jnp.float32),
        pltpu.VMEM((_CHUNK,), jnp.float32),
        pltpu.SemaphoreType.DMA,
        pltpu.SemaphoreType.DMA,
        pltpu.SemaphoreType.DMA,
        pltpu.SemaphoreType.DMA,
        pltpu.SemaphoreType.DMA,
        pltpu.SemaphoreType.DMA,
        pltpu.SemaphoreType.DMA,
        pltpu.SemaphoreType.DMA,
        pltpu.SemaphoreType.DMA,
        pltpu.SemaphoreType.DMA,
    ],
)(_als_body)


def kernel(user_id, item_id, u, i, u_bias, i_bias):
    return _als(user_id.astype(jnp.int32), item_id.astype(jnp.int32),
                u.T, i.T, u_bias, i_bias)
